# baseline (speedup 1.0000x reference)
import functools

import jax
import jax.numpy as jnp
from jax import lax
from jax.experimental import pallas as pl
from jax.experimental.pallas import tpu as pltpu
from jax.experimental.pallas import tpu_sc as plsc

GX, GY, GZ, D = 64, 64, 64, 32
V = GX * GY * GZ
N = 262144

NC, NS, L = 2, 16, 16
NW = NC * NS
BPW = N // NW
C = 512
NCH = BPW // C
CT = C // 128
NT = N // 128
FT = D // 8
TSZ = C * D

_mesh = plsc.VectorSubcoreMesh(core_axis_name="c", subcore_axis_name="s")


@functools.partial(
    pl.kernel,
    mesh=_mesh,
    out_type=jax.ShapeDtypeStruct((N * D,), jnp.float32),
    compiler_params=pltpu.CompilerParams(
        needs_layout_passes=False, use_tc_tiling_on_sc=False
    ),
    scratch_types=[
        pltpu.VMEM((2, C * 3), jnp.int32),
        pltpu.VMEM((2, C), jnp.int32),
        pltpu.VMEM((2, C, D), jnp.float32),
        pltpu.VMEM((2, TSZ), jnp.float32),
        pltpu.SemaphoreType.DMA,
        pltpu.SemaphoreType.DMA,
        pltpu.SemaphoreType.DMA,
        pltpu.SemaphoreType.DMA,
    ],
)
def _sc_gather(
    table_hbm, idx_hbm, out_hbm, idx_v, lin_v, rows_v, trans_v,
    sem_g0, sem_g1, sem_o0, sem_o1,
):
    wid = lax.axis_index("s") * NC + lax.axis_index("c")
    base = wid * BPW
    sems_g = (sem_g0, sem_g1)
    sems_o = (sem_o0, sem_o1)

    lane = lax.iota(jnp.int32, L)
    f0 = lane
    f1 = lane + 16
    cvec0 = ((f0 >> 3) * (CT * 1024)) + ((f0 & 7) << 7)
    cvec1 = ((f1 >> 3) * (CT * 1024)) + ((f1 & 7) << 7)

    def load_and_linearize(g, b):
        off = base + g * C
        pltpu.sync_copy(idx_hbm.at[pl.ds(off, C)], idx_v.at[b, pl.ds(0, C)])
        pltpu.sync_copy(idx_hbm.at[pl.ds(N + off, C)], idx_v.at[b, pl.ds(C, C)])
        pltpu.sync_copy(
            idx_hbm.at[pl.ds(2 * N + off, C)], idx_v.at[b, pl.ds(2 * C, C)]
        )

        def linearize(i, carry):
            sl = pl.ds(i * L, L)
            x = idx_v[b, pl.ds(i * L, L)]
            y = idx_v[b, pl.ds(C + i * L, L)]
            z = idx_v[b, pl.ds(2 * C + i * L, L)]
            s = (x << 6) + y
            lin_v[b, sl] = ((s >> 2) << 8) + (z << 2) + (s & 3)
            return carry

        lax.fori_loop(0, C // L, linearize, 0, unroll=4)

    def start_gather(b):
        return pltpu.async_copy(
            table_hbm.at[lin_v.at[b]], rows_v.at[b], sems_g[b]
        )

    def transpose_chunk(b):
        def transpose(i, carry):
            rows16 = (i << 4) + lane
            pbase = ((i >> 3) << 10) + ((i & 7) << 4)
            for f in range(D):
                val = plsc.load_gather(
                    rows_v.at[b], [rows16, jnp.full((L,), f, jnp.int32)]
                )
                dst = (f >> 3) * (CT * 1024) + (f & 7) * 128
                trans_v[b, pl.ds(dst + pbase, L)] = val
            return carry

        lax.fori_loop(0, C // L, transpose, 0)

    def start_out(g, b):
        off = base + g * C
        ntq = off >> 7
        handles = []
        for ft in range(FT):
            handles.append(
                pltpu.async_copy(
                    trans_v.at[b, pl.ds(ft * (CT * 1024), CT * 1024)],
                    out_hbm.at[pl.ds(ft * (NT * 1024) + ntq * 1024, CT * 1024)],
                    sems_o[b],
                )
            )
        return handles

    load_and_linearize(0, 0)
    gh = {0: start_gather(0)}
    oh = {}
    for g in range(NCH):
        b = g & 1
        nb = b ^ 1
        if g + 1 < NCH:
            load_and_linearize(g + 1, nb)
        gh.pop(g).wait()
        if g + 1 < NCH:
            gh[g + 1] = start_gather(nb)
        if g - 2 in oh:
            for h in oh.pop(g - 2):
                h.wait()
        transpose_chunk(b)
        oh[g] = start_out(g, b)
    for hs in oh.values():
        for h in hs:
            h.wait()


def _tc_detile_body(t2_ref, out_ref):
    x = t2_ref[...]
    xr = x.reshape(16, 128, 64)
    out_ref[...] = jnp.transpose(xr, (0, 2, 1))


_tc_detile = pl.pallas_call(
    _tc_detile_body,
    grid=(64,),
    in_specs=[pl.BlockSpec((64, 32, 64), lambda i: (i, 0, 0))],
    out_specs=pl.BlockSpec((16, 64, 128), lambda i: (i, 0, 0)),
    out_shape=jax.ShapeDtypeStruct((1024, 64, 128), jnp.float32),
)


def kernel(in_tensor, in_index):
    t2 = jnp.transpose(in_tensor, (0, 1, 3, 2)).reshape(4096, 32, 64)
    table = _tc_detile(t2).reshape(V, D)
    idx = in_index.astype(jnp.int32).T.reshape(3 * N)
    out_flat = _sc_gather(table, idx)
    out4 = out_flat.reshape(FT, NT, 8, 128)
    return out4.transpose(1, 3, 0, 2).reshape(N, D)


# speedup vs baseline: 1.2742x; 1.2742x over previous
import functools

import jax
import jax.numpy as jnp
from jax import lax
from jax.experimental import pallas as pl
from jax.experimental.pallas import tpu as pltpu
from jax.experimental.pallas import tpu_sc as plsc

GX, GY, GZ, D = 64, 64, 64, 32
V = GX * GY * GZ
N = 262144

NC, NS, L = 2, 16, 16
NW = NC * NS
BPW = N // NW
C = 1024
NCH = BPW // C

_mesh = plsc.VectorSubcoreMesh(core_axis_name="c", subcore_axis_name="s")


@functools.partial(
    pl.kernel,
    mesh=_mesh,
    out_type=jax.ShapeDtypeStruct((N, D), jnp.float32),
    compiler_params=pltpu.CompilerParams(
        needs_layout_passes=False, use_tc_tiling_on_sc=False
    ),
    scratch_types=[
        pltpu.VMEM((2, C * 3), jnp.int32),
        pltpu.VMEM((2, C), jnp.int32),
        pltpu.VMEM((2, C, D), jnp.float32),
        pltpu.SemaphoreType.DMA,
        pltpu.SemaphoreType.DMA,
        pltpu.SemaphoreType.DMA,
        pltpu.SemaphoreType.DMA,
    ],
)
def _sc_gather(
    table_hbm, idx_hbm, out_hbm, idx_v, lin_v, rows_v,
    sem_g0, sem_g1, sem_o0, sem_o1,
):
    wid = lax.axis_index("s") * NC + lax.axis_index("c")
    base = wid * BPW
    sems_g = (sem_g0, sem_g1)
    sems_o = (sem_o0, sem_o1)

    def load_and_linearize(g, b):
        off = base + g * C
        pltpu.sync_copy(idx_hbm.at[pl.ds(off, C)], idx_v.at[b, pl.ds(0, C)])
        pltpu.sync_copy(idx_hbm.at[pl.ds(N + off, C)], idx_v.at[b, pl.ds(C, C)])
        pltpu.sync_copy(
            idx_hbm.at[pl.ds(2 * N + off, C)], idx_v.at[b, pl.ds(2 * C, C)]
        )

        def linearize(i, carry):
            sl = pl.ds(i * L, L)
            x = idx_v[b, pl.ds(i * L, L)]
            y = idx_v[b, pl.ds(C + i * L, L)]
            z = idx_v[b, pl.ds(2 * C + i * L, L)]
            s = (x << 6) + y
            lin_v[b, sl] = ((s >> 2) << 8) + (z << 2) + (s & 3)
            return carry

        lax.fori_loop(0, C // L, linearize, 0, unroll=4)

    def start_gather(b):
        return pltpu.async_copy(
            table_hbm.at[lin_v.at[b]], rows_v.at[b], sems_g[b]
        )

    def start_out(g, b):
        off = base + g * C
        return pltpu.async_copy(
            rows_v.at[b], out_hbm.at[pl.ds(off, C)], sems_o[b]
        )

    load_and_linearize(0, 0)
    gh = {0: start_gather(0)}
    oh = {}
    for g in range(NCH):
        b = g & 1
        nb = b ^ 1
        if g + 1 < NCH:
            load_and_linearize(g + 1, nb)
            if g - 1 in oh:
                oh.pop(g - 1).wait()
            gh.pop(g).wait()
            gh[g + 1] = start_gather(nb)
        else:
            gh.pop(g).wait()
        oh[g] = start_out(g, b)
    for h in oh.values():
        h.wait()


def _tc_detile_body(t2_ref, out_ref):
    x = t2_ref[...]
    xr = x.reshape(16, 128, 64)
    out_ref[...] = jnp.transpose(xr, (0, 2, 1))


_tc_detile = pl.pallas_call(
    _tc_detile_body,
    grid=(64,),
    in_specs=[pl.BlockSpec((64, 32, 64), lambda i: (i, 0, 0))],
    out_specs=pl.BlockSpec((16, 64, 128), lambda i: (i, 0, 0)),
    out_shape=jax.ShapeDtypeStruct((1024, 64, 128), jnp.float32),
)


def kernel(in_tensor, in_index):
    t2 = jnp.transpose(in_tensor, (0, 1, 3, 2)).reshape(4096, 32, 64)
    table = _tc_detile(t2).reshape(V, D)
    idx = in_index.astype(jnp.int32).T.reshape(3 * N)
    return _sc_gather(table, idx)
